# Initial kernel scaffold; baseline (speedup 1.0000x reference)
import jax
import jax.numpy as jnp
from jax.experimental import pallas as pl


def kernel(output, target):
    raise NotImplementedError("write your pallas kernel here")


# speedup vs baseline: 1122.3437x; 1122.3437x over previous
import numpy as np
import jax
import jax.numpy as jnp
from jax import lax
from jax.experimental import pallas as pl

_ANCHORS = np.array(
    [0.57273, 0.677385, 1.87446, 2.06253, 3.33843, 5.47434,
     7.88282, 3.52778, 9.77052, 9.16828], dtype=np.float32)
_AW = _ANCHORS[0::2]
_AH = _ANCHORS[1::2]
_NA = 5
_NC = 80
_NH = 19
_NW = 19
_NPIX = _NH * _NW
_NT = 50
_THRESH = 0.6
_OBJ = 5.0


def _sig(v):
    return 1.0 / (1.0 + jnp.exp(-v))


def _iou(b1x, b1y, b1w, b1h, b2x, b2y, b2w, b2h):
    b1x1 = b1x - b1w / 2.0
    b1x2 = b1x + b1w / 2.0
    b1y1 = b1y - b1h / 2.0
    b1y2 = b1y + b1h / 2.0
    b2x1 = b2x - b2w / 2.0
    b2x2 = b2x + b2w / 2.0
    b2y1 = b2y - b2h / 2.0
    b2y2 = b2y + b2h / 2.0
    mx = jnp.minimum(b1x1, b2x1)
    Mx = jnp.maximum(b1x2, b2x2)
    my = jnp.minimum(b1y1, b2y1)
    My = jnp.maximum(b1y2, b2y2)
    cw = b1w + b2w - (Mx - mx)
    ch = b1h + b2h - (My - my)
    carea = jnp.where((cw <= 0) | (ch <= 0), 0.0, cw * ch)
    return carea / (b1w * b1h + b2w * b2h - carea)


def _yolo_kernel(o_ref, trow_ref, tcol_ref, grid_ref, out_ref):
    f32 = jnp.float32
    i32 = jnp.int32

    cls_r = trow_ref[0, 0:1, :]
    xs_r = trow_ref[0, 1:2, :]
    gx_r = xs_r * float(_NW)
    gy_r = trow_ref[0, 2:3, :] * float(_NH)
    gw_r = trow_ref[0, 3:4, :] * float(_NW)
    gh_r = trow_ref[0, 4:5, :] * float(_NH)

    xs_c = tcol_ref[0, :, 1:2]
    gx_c = xs_c * float(_NW)
    gy_c = tcol_ref[0, :, 2:3] * float(_NH)
    gw_c = tcol_ref[0, :, 3:4] * float(_NW)
    gh_c = tcol_ref[0, :, 4:5] * float(_NH)

    io_r = lax.broadcasted_iota(i32, (_NT, _NT), 0)
    io_c = lax.broadcasted_iota(i32, (_NT, _NT), 1)
    nz_r = (xs_r != 0.0)
    nz_c = (xs_c != 0.0)
    valid_c = jnp.min(jnp.where(io_c <= io_r, nz_r.astype(f32), 1.0),
                      axis=1, keepdims=True) > 0.5
    valid_r = jnp.min(jnp.where(io_r <= io_c, nz_c.astype(f32), 1.0),
                      axis=0, keepdims=True) > 0.5

    def best_anchor(gw, gh):
        zero = jnp.zeros_like(gw)
        best_v = jnp.full_like(gw, -jnp.inf)
        best_n = jnp.zeros(gw.shape, i32)
        for a in range(_NA):
            v = _iou(zero, zero, jnp.full_like(gw, _AW[a]),
                     jnp.full_like(gw, _AH[a]), zero, zero, gw, gh)
            take = v > best_v
            best_v = jnp.where(take, v, best_v)
            best_n = jnp.where(take, jnp.full(gw.shape, a, i32), best_n)
        return best_n

    n_r = best_anchor(gw_r, gh_r)
    n_c = best_anchor(gw_c, gh_c)

    gi_r = gx_r.astype(i32)
    gj_r = gy_r.astype(i32)
    gi_c = gx_c.astype(i32)
    gj_c = gy_c.astype(i32)
    p_r = gj_r * _NW + gi_r
    p_c = gj_c * _NW + gi_c
    cell_r = n_r * _NPIX + p_r
    cell_c = n_c * _NPIX + p_c

    ow_c = jnp.max(jnp.where((cell_r == cell_c) & nzv(valid_r) & (io_c > io_r),
                             1.0, 0.0), axis=1, keepdims=True)
    winner_c = valid_c & (ow_c < 0.5)
    ow_r = jnp.max(jnp.where((cell_c == cell_r) & nzv(valid_c) & (io_r > io_c),
                             1.0, 0.0), axis=0, keepdims=True)
    winner_r = valid_r & (ow_r < 0.5)

    gxgrid = grid_ref[0:1, :]
    gygrid = grid_ref[1:2, :]

    p_io = lax.broadcasted_iota(i32, (_NT, _NPIX), 1)

    sum_xy = jnp.float32(0.0)
    sum_wh = jnp.float32(0.0)
    sum_conf = jnp.float32(0.0)
    zcol = jnp.zeros((_NT, 1), f32)
    g_sigx = zcol; g_sigy = zcol; g_w = zcol; g_h = zcol
    g_conf = zcol; g_pw = zcol; g_ph = zcol; g_cur = zcol
    Lg = jnp.zeros((_NC, _NT), f32)

    for a in range(_NA):
        base = a * (5 + _NC)
        x_a = o_ref[0, base + 0:base + 1, :]
        y_a = o_ref[0, base + 1:base + 2, :]
        w_a = o_ref[0, base + 2:base + 3, :]
        h_a = o_ref[0, base + 3:base + 4, :]
        c_a = o_ref[0, base + 4:base + 5, :]
        sigx = _sig(x_a); sigy = _sig(y_a); sigc = _sig(c_a)
        pxc = sigx + gxgrid
        pyc = sigy + gygrid
        pw = jnp.exp(w_a) * _AW[a]
        ph = jnp.exp(h_a) * _AH[a]

        sum_xy += jnp.sum((sigx - 0.5) ** 2) + jnp.sum((sigy - 0.5) ** 2)
        sum_wh += jnp.sum(w_a * w_a) + jnp.sum(h_a * h_a)

        ioum = _iou(pxc, pyc, pw, ph, gx_c, gy_c, gw_c, gh_c)
        cur_a = jnp.max(jnp.where(valid_c, ioum, 0.0), axis=0,
                        keepdims=True)
        mask0 = jnp.where(cur_a > _THRESH, 0.0, 1.0)
        sum_conf += jnp.sum(mask0 * sigc * sigc)

        mskf = jnp.where((p_io == p_c) & (n_c == a), 1.0, 0.0)
        g_sigx += jnp.sum(mskf * sigx, axis=1, keepdims=True)
        g_sigy += jnp.sum(mskf * sigy, axis=1, keepdims=True)
        g_w += jnp.sum(mskf * w_a, axis=1, keepdims=True)
        g_h += jnp.sum(mskf * h_a, axis=1, keepdims=True)
        g_conf += jnp.sum(mskf * sigc, axis=1, keepdims=True)
        g_pw += jnp.sum(mskf * pw, axis=1, keepdims=True)
        g_ph += jnp.sum(mskf * ph, axis=1, keepdims=True)
        g_cur += jnp.sum(mskf * cur_a, axis=1, keepdims=True)

        cls_a = o_ref[0, base + 5:base + 5 + _NC, :]
        Lg += lax.dot_general(cls_a, mskf, (((1,), (1,)), ((), ())),
                              preferred_element_type=f32)

    gi_f = gi_c.astype(f32)
    gj_f = gj_c.astype(f32)
    txw = gx_c - gi_f
    tyw = gy_c - gj_f
    aw_at = jnp.zeros((_NT, 1), f32)
    ah_at = jnp.zeros((_NT, 1), f32)
    for a in range(_NA):
        aw_at = jnp.where(n_c == a, _AW[a], aw_at)
        ah_at = jnp.where(n_c == a, _AH[a], ah_at)
    tww = jnp.log(gw_c / aw_at)
    thw = jnp.log(gh_c / ah_at)
    pxc_at = g_sigx + gi_f
    pyc_at = g_sigy + gj_f
    iou_at = _iou(gx_c, gy_c, gw_c, gh_c, pxc_at, pyc_at, g_pw, g_ph)
    mask0_at = jnp.where(g_cur > _THRESH, 0.0, 1.0)

    corr = ((g_sigx - txw) ** 2 - (g_sigx - 0.5) ** 2
            + (g_sigy - tyw) ** 2 - (g_sigy - 0.5) ** 2
            + (g_w - tww) ** 2 - g_w * g_w
            + (g_h - thw) ** 2 - g_h * g_h
            + _OBJ * (g_conf - iou_at) ** 2 - mask0_at * g_conf * g_conf)
    corr_sum = jnp.sum(jnp.where(winner_c, corr, 0.0))

    cint = cls_r.astype(i32)
    c_io = lax.broadcasted_iota(i32, (_NC, _NT), 0)
    pick = jnp.sum(jnp.where(c_io == cint, Lg, 0.0), axis=0, keepdims=True)
    m = jnp.max(Lg, axis=0, keepdims=True)
    lse = m + jnp.log(jnp.sum(jnp.exp(Lg - m), axis=0, keepdims=True))
    nll = lse - pick
    cls_sum = jnp.sum(jnp.where(winner_r, nll, 0.0))

    total = (sum_xy + sum_wh + sum_conf + corr_sum) * 0.5 + cls_sum
    out_ref[0] = jnp.full((1, 1), total, jnp.float32)


def nzv(b):
    return b


def _grid_consts():
    p = np.arange(_NPIX)
    return np.stack([(p % _NW).astype(np.float32),
                     (p // _NW).astype(np.float32)], axis=0)


def kernel(output, target):
    nB = output.shape[0]
    o = output.reshape(nB, _NA * (5 + _NC), _NPIX)
    tcol = target.reshape(nB, _NT, 5)
    trow = tcol.transpose(0, 2, 1)
    gridc = jnp.asarray(_grid_consts())

    partial = pl.pallas_call(
        _yolo_kernel,
        grid=(nB,),
        in_specs=[
            pl.BlockSpec((1, _NA * (5 + _NC), _NPIX), lambda b: (b, 0, 0)),
            pl.BlockSpec((1, 5, _NT), lambda b: (b, 0, 0)),
            pl.BlockSpec((1, _NT, 5), lambda b: (b, 0, 0)),
            pl.BlockSpec((2, _NPIX), lambda b: (0, 0)),
        ],
        out_specs=pl.BlockSpec((1, 1, 1), lambda b: (b, 0, 0)),
        out_shape=jax.ShapeDtypeStruct((nB, 1, 1), jnp.float32),
    )(o, trow, tcol, gridc)
    return jnp.sum(partial)
